# baseline (speedup 1.0000x reference)
import functools

import jax
import jax.numpy as jnp
from jax import lax
from jax.experimental import pallas as pl
from jax.experimental.pallas import tpu as pltpu
from jax.experimental.pallas import tpu_sc as plsc

N_NODES = 10000
D = 128
N_EDGES = 320000

NC = 2
NS = 16
NW = NC * NS
CH = 64
NBUF = 2
HALF = 80
C0_CHUNKS = 320
PADDED_E = NS * C0_CHUNKS * CH
NPAD = 10240
ROWS_PER_TILE = NPAD // NS
DUMMY_DST = N_NODES


def _sc_aggregate(src2d, dst2d, x, zeros):
    mesh = plsc.VectorSubcoreMesh(core_axis_name="c", subcore_axis_name="s")

    @functools.partial(
        pl.kernel,
        mesh=mesh,
        out_type=jax.ShapeDtypeStruct((NPAD, D), jnp.float32),
        scratch_types=[
            pltpu.VMEM((HALF, CH), jnp.int32),
            pltpu.VMEM((HALF, CH), jnp.int32),
            pltpu.VMEM((NBUF, CH, D), jnp.float32),
            pltpu.VMEM_SHARED((NPAD, D), jnp.float32),
        ] + [pltpu.SemaphoreType.DMA] * NBUF,
    )
    def agg(src_hbm, dst_hbm, x_hbm, zeros_hbm, out_hbm,
            src_v, dst_v, rows_v, acc, *gsems):
        c = lax.axis_index("c")
        s = lax.axis_index("s")

        @pl.when(c == 0)
        def _():
            pltpu.sync_copy(zeros_hbm,
                            acc.at[pl.ds(s * ROWS_PER_TILE, ROWS_PER_TILE)])

        plsc.subcore_barrier()

        def fire_gather(buf, chunk):
            pltpu.async_copy(x_hbm.at[src_v.at[chunk]],
                             rows_v.at[buf], gsems[buf])

        def wait_gather(buf):
            pltpu.make_async_copy(x_hbm.at[src_v.at[0]],
                                  rows_v.at[buf], gsems[buf]).wait()

        def scatter(buf, chunk):
            pltpu.sync_copy(rows_v.at[buf], acc.at[dst_v.at[chunk]], add=True)

        def run_half(chunk0):
            pltpu.sync_copy(src_hbm.at[pl.ds(chunk0, HALF)], src_v)
            pltpu.sync_copy(dst_hbm.at[pl.ds(chunk0, HALF)], dst_v)

            for b in range(NBUF):
                fire_gather(b, b)

            def step(g, carry):
                base = NBUF * g
                for b in range(NBUF):
                    wait_gather(b)
                    scatter(b, base + b)
                    fire_gather(b, jnp.minimum(base + NBUF + b, HALF - 1))
                return carry

            lax.fori_loop(0, HALF // NBUF, step, 0)
            for b in range(NBUF):
                wait_gather(b)

        @pl.when(c == 0)
        def _():
            for h in range(C0_CHUNKS // HALF):
                run_half(s * C0_CHUNKS + h * HALF)

        plsc.subcore_barrier()

        @pl.when(c == 0)
        def _():
            pltpu.sync_copy(acc.at[pl.ds(s * ROWS_PER_TILE, ROWS_PER_TILE)],
                            out_hbm.at[pl.ds(s * ROWS_PER_TILE, ROWS_PER_TILE)])

    return agg(src2d, dst2d, x, zeros)


def _tc_finish_body(agg_ref, w_ref, b_ref, o_ref):
    y = jnp.dot(agg_ref[...], w_ref[...], preferred_element_type=jnp.float32)
    o_ref[...] = jnp.maximum(y + b_ref[...], 0.0)


def _tc_finish(partials, W, b):
    rb = 1000
    return pl.pallas_call(
        _tc_finish_body,
        grid=(N_NODES // rb,),
        in_specs=[
            pl.BlockSpec((rb, D), lambda i: (i, 0)),
            pl.BlockSpec((D, D), lambda i: (0, 0)),
            pl.BlockSpec((1, D), lambda i: (0, 0)),
        ],
        out_specs=pl.BlockSpec((rb, D), lambda i: (i, 0)),
        out_shape=jax.ShapeDtypeStruct((N_NODES, D), jnp.float32),
    )(partials, W, b.reshape(1, D))


@jax.jit
def kernel(x, edge_index, W, b):
    src = edge_index[0].astype(jnp.int32)
    dst = edge_index[1].astype(jnp.int32)
    pad = PADDED_E - N_EDGES
    src = jnp.concatenate([src, jnp.zeros((pad,), jnp.int32)])
    dst = jnp.concatenate(
        [dst, DUMMY_DST + jnp.arange(pad, dtype=jnp.int32) % (NPAD - N_NODES)])
    src2d = src.reshape(PADDED_E // CH, CH)
    dst2d = dst.reshape(PADDED_E // CH, CH)
    zeros = jnp.zeros((ROWS_PER_TILE, D), jnp.float32)

    partials = _sc_aggregate(src2d, dst2d, x, zeros)
    out = _tc_finish(partials, W, b)
    return (out, edge_index)


# speedup vs baseline: 1.5284x; 1.5284x over previous
import functools

import numpy as np

import jax
import jax.numpy as jnp
from jax import lax
from jax.experimental import pallas as pl
from jax.experimental.pallas import tpu as pltpu
from jax.experimental.pallas import tpu_sc as plsc

N_NODES = 10000
D = 128
N_EDGES = 320000

NC = 2
NS = 16
NW = NC * NS
CH = 64
NBUF = 2
HALF = 80
C0_CHUNKS = 320
PADDED_E = NS * C0_CHUNKS * CH
NPAD = 10240
ROWS_PER_TILE = NPAD // NS
DUMMY_DST = N_NODES


def _sc_aggregate(src2d, dst2d, x, zeros):
    mesh = plsc.VectorSubcoreMesh(core_axis_name="c", subcore_axis_name="s")

    @functools.partial(
        pl.kernel,
        mesh=mesh,
        out_type=jax.ShapeDtypeStruct((NPAD, D), jnp.float32),
        scratch_types=[
            pltpu.VMEM((HALF, CH), jnp.int32),
            pltpu.VMEM((HALF, CH), jnp.int32),
            pltpu.VMEM((NBUF, CH, D), jnp.float32),
            pltpu.VMEM_SHARED((NPAD, D), jnp.float32),
        ] + [pltpu.SemaphoreType.DMA] * NBUF,
    )
    def agg(src_hbm, dst_hbm, x_hbm, zeros_hbm, out_hbm,
            src_v, dst_v, rows_v, acc, *gsems):
        c = lax.axis_index("c")
        s = lax.axis_index("s")

        @pl.when(c == 0)
        def _():
            pltpu.sync_copy(zeros_hbm,
                            acc.at[pl.ds(s * ROWS_PER_TILE, ROWS_PER_TILE)])

        plsc.subcore_barrier()

        def fire_gather(buf, chunk):
            pltpu.async_copy(x_hbm.at[src_v.at[chunk]],
                             rows_v.at[buf], gsems[buf])

        def wait_gather(buf):
            pltpu.make_async_copy(x_hbm.at[src_v.at[0]],
                                  rows_v.at[buf], gsems[buf]).wait()

        def scatter(buf, chunk):
            pltpu.sync_copy(rows_v.at[buf], acc.at[dst_v.at[chunk]], add=True)

        def run_half(chunk0):
            pltpu.sync_copy(src_hbm.at[pl.ds(chunk0, HALF)], src_v)
            pltpu.sync_copy(dst_hbm.at[pl.ds(chunk0, HALF)], dst_v)

            for b in range(NBUF):
                fire_gather(b, b)

            def step(g, carry):
                base = NBUF * g
                for b in range(NBUF):
                    wait_gather(b)
                    scatter(b, base + b)
                    fire_gather(b, jnp.minimum(base + NBUF + b, HALF - 1))
                return carry

            lax.fori_loop(0, HALF // NBUF, step, 0)
            for b in range(NBUF):
                wait_gather(b)

        @pl.when(c == 0)
        def _():
            for h in range(C0_CHUNKS // HALF):
                run_half(s * C0_CHUNKS + h * HALF)

        plsc.subcore_barrier()

        @pl.when(c == 0)
        def _():
            pltpu.sync_copy(acc.at[pl.ds(s * ROWS_PER_TILE, ROWS_PER_TILE)],
                            out_hbm.at[pl.ds(s * ROWS_PER_TILE, ROWS_PER_TILE)])

    return agg(src2d, dst2d, x, zeros)


def _tc_finish_body(agg_ref, w_ref, b_ref, o_ref):
    y = jnp.dot(agg_ref[...], w_ref[...], preferred_element_type=jnp.float32)
    o_ref[...] = jnp.maximum(y + b_ref[...], 0.0)


def _tc_finish(partials, W, b):
    rb = 1000
    return pl.pallas_call(
        _tc_finish_body,
        grid=(N_NODES // rb,),
        in_specs=[
            pl.BlockSpec((rb, D), lambda i: (i, 0)),
            pl.BlockSpec((D, D), lambda i: (0, 0)),
            pl.BlockSpec((1, D), lambda i: (0, 0)),
        ],
        out_specs=pl.BlockSpec((rb, D), lambda i: (i, 0)),
        out_shape=jax.ShapeDtypeStruct((N_NODES, D), jnp.float32),
    )(partials, W, b.reshape(1, D))


@jax.jit
def kernel(x, edge_index, W, b):
    src = edge_index[0].astype(jnp.int32)
    dst = edge_index[1].astype(jnp.int32)
    pad = PADDED_E - N_EDGES
    src = jnp.concatenate([src, jnp.zeros((pad,), jnp.int32)])
    dst = jnp.concatenate(
        [dst, DUMMY_DST + jnp.arange(pad, dtype=jnp.int32) % (NPAD - N_NODES)])
    src2d = src.reshape(PADDED_E // CH, CH)
    dst2d = dst.reshape(PADDED_E // CH, CH)
    n_chunks = PADDED_E // CH
    n_real = N_EDGES // CH
    n_dummy = n_chunks - n_real
    dummy_slots = (np.arange(n_dummy) * n_chunks) // n_dummy
    real_slots = np.setdiff1d(np.arange(n_chunks), dummy_slots)
    perm = np.empty(n_chunks, np.int32)
    perm[dummy_slots] = n_real + np.arange(n_dummy)
    perm[real_slots] = np.arange(n_real)
    src2d = jnp.take(src2d, perm, axis=0)
    dst2d = jnp.take(dst2d, perm, axis=0)
    zeros = jnp.zeros((ROWS_PER_TILE, D), jnp.float32)

    partials = _sc_aggregate(src2d, dst2d, x, zeros)
    out = _tc_finish(partials, W, b)
    return (out, edge_index)


# speedup vs baseline: 1.6263x; 1.0641x over previous
import functools

import numpy as np

import jax
import jax.numpy as jnp
from jax import lax
from jax.experimental import pallas as pl
from jax.experimental.pallas import tpu as pltpu
from jax.experimental.pallas import tpu_sc as plsc

N_NODES = 10000
D = 128
N_EDGES = 320000

NC = 2
NS = 16
NW = NC * NS
CH = 64
NBUF = 4
HALF = 40
C0_CHUNKS = 320
PADDED_E = NS * C0_CHUNKS * CH
NPAD = 10240
ROWS_PER_TILE = NPAD // NS
DUMMY_DST = N_NODES


def _sc_aggregate(src2d, dst2d, x, zeros):
    mesh = plsc.VectorSubcoreMesh(core_axis_name="c", subcore_axis_name="s")

    @functools.partial(
        pl.kernel,
        mesh=mesh,
        out_type=jax.ShapeDtypeStruct((NPAD, D), jnp.float32),
        scratch_types=[
            pltpu.VMEM((HALF, CH), jnp.int32),
            pltpu.VMEM((HALF, CH), jnp.int32),
            pltpu.VMEM((NBUF, CH, D), jnp.float32),
            pltpu.VMEM_SHARED((NPAD, D), jnp.float32),
        ] + [pltpu.SemaphoreType.DMA] * NBUF,
    )
    def agg(src_hbm, dst_hbm, x_hbm, zeros_hbm, out_hbm,
            src_v, dst_v, rows_v, acc, *gsems):
        c = lax.axis_index("c")
        s = lax.axis_index("s")

        @pl.when(c == 0)
        def _():
            pltpu.sync_copy(zeros_hbm,
                            acc.at[pl.ds(s * ROWS_PER_TILE, ROWS_PER_TILE)])

        plsc.subcore_barrier()

        def fire_gather(buf, chunk):
            pltpu.async_copy(x_hbm.at[src_v.at[chunk]],
                             rows_v.at[buf], gsems[buf])

        def wait_gather(buf):
            pltpu.make_async_copy(x_hbm.at[src_v.at[0]],
                                  rows_v.at[buf], gsems[buf]).wait()

        def scatter(buf, chunk):
            pltpu.sync_copy(rows_v.at[buf], acc.at[dst_v.at[chunk]], add=True)

        def run_half(chunk0):
            pltpu.sync_copy(src_hbm.at[pl.ds(chunk0, HALF)], src_v)
            pltpu.sync_copy(dst_hbm.at[pl.ds(chunk0, HALF)], dst_v)

            for b in range(NBUF):
                fire_gather(b, b)

            def step(g, carry):
                base = NBUF * g
                for b in range(NBUF):
                    wait_gather(b)
                    scatter(b, base + b)
                    fire_gather(b, jnp.minimum(base + NBUF + b, HALF - 1))
                return carry

            lax.fori_loop(0, HALF // NBUF, step, 0)
            for b in range(NBUF):
                wait_gather(b)

        @pl.when(c == 0)
        def _():
            for h in range(C0_CHUNKS // HALF):
                run_half(s * C0_CHUNKS + h * HALF)

        plsc.subcore_barrier()

        @pl.when(c == 0)
        def _():
            pltpu.sync_copy(acc.at[pl.ds(s * ROWS_PER_TILE, ROWS_PER_TILE)],
                            out_hbm.at[pl.ds(s * ROWS_PER_TILE, ROWS_PER_TILE)])

    return agg(src2d, dst2d, x, zeros)


def _tc_finish_body(agg_ref, w_ref, b_ref, o_ref):
    y = jnp.dot(agg_ref[...], w_ref[...], preferred_element_type=jnp.float32)
    o_ref[...] = jnp.maximum(y + b_ref[...], 0.0)


def _tc_finish(partials, W, b):
    rb = 1000
    return pl.pallas_call(
        _tc_finish_body,
        grid=(N_NODES // rb,),
        in_specs=[
            pl.BlockSpec((rb, D), lambda i: (i, 0)),
            pl.BlockSpec((D, D), lambda i: (0, 0)),
            pl.BlockSpec((1, D), lambda i: (0, 0)),
        ],
        out_specs=pl.BlockSpec((rb, D), lambda i: (i, 0)),
        out_shape=jax.ShapeDtypeStruct((N_NODES, D), jnp.float32),
    )(partials, W, b.reshape(1, D))


@jax.jit
def kernel(x, edge_index, W, b):
    src = edge_index[0].astype(jnp.int32)
    dst = edge_index[1].astype(jnp.int32)
    pad = PADDED_E - N_EDGES
    src = jnp.concatenate([src, jnp.zeros((pad,), jnp.int32)])
    dst = jnp.concatenate(
        [dst, DUMMY_DST + jnp.arange(pad, dtype=jnp.int32) % (NPAD - N_NODES)])
    src2d = src.reshape(PADDED_E // CH, CH)
    dst2d = dst.reshape(PADDED_E // CH, CH)
    n_chunks = PADDED_E // CH
    n_real = N_EDGES // CH
    n_dummy = n_chunks - n_real
    dummy_slots = (np.arange(n_dummy) * n_chunks) // n_dummy
    real_slots = np.setdiff1d(np.arange(n_chunks), dummy_slots)
    perm = np.empty(n_chunks, np.int32)
    perm[dummy_slots] = n_real + np.arange(n_dummy)
    perm[real_slots] = np.arange(n_real)
    src2d = jnp.take(src2d, perm, axis=0)
    dst2d = jnp.take(dst2d, perm, axis=0)
    zeros = jnp.zeros((ROWS_PER_TILE, D), jnp.float32)

    partials = _sc_aggregate(src2d, dst2d, x, zeros)
    out = _tc_finish(partials, W, b)
    return (out, edge_index)
